# baseline (speedup 1.0000x reference)
import functools

import jax
import jax.numpy as jnp
from jax import lax
from jax.experimental import pallas as pl
from jax.experimental.pallas import tpu as pltpu
from jax.experimental.pallas import tpu_sc as plsc


@functools.lru_cache(maxsize=None)
def _make_lookup(S0, S1, V, D):
    info = plsc.get_sparse_core_info()
    NC, NS, VL = info.num_cores, info.num_subcores, info.num_lanes
    NW = NC * NS
    LT = 128
    DR = D // 8
    n_lt = S0 // LT
    lt_per_w = n_lt // NW
    n_units = S1 * lt_per_w
    b_per_w = n_units * LT
    assert S0 % (LT * NW) == 0 and D % 8 == 0 and n_units % 2 == 0

    mesh = plsc.VectorSubcoreMesh(core_axis_name="c", subcore_axis_name="s")

    @functools.partial(
        pl.kernel,
        mesh=mesh,
        compiler_params=pltpu.CompilerParams(
            use_tc_tiling_on_sc=False, needs_layout_passes=False
        ),
        out_type=jax.ShapeDtypeStruct((S1, DR, n_lt, 8, LT), jnp.float32),
        scratch_types=[
            pltpu.VMEM((b_per_w,), jnp.int32),
            pltpu.VMEM((LT, D), jnp.float32),
            pltpu.VMEM((LT, D), jnp.float32),
            pltpu.VMEM((DR, 8, LT + 1), jnp.float32),
            pltpu.VMEM((DR, 8, LT + 1), jnp.float32),
            pltpu.SemaphoreType.DMA,
            pltpu.SemaphoreType.DMA,
        ],
    )
    def lookup(idx_hbm, table_hbm, out_hbm, idx_v, r0, r1, t0, t1, gsem, wsem):
        wid = lax.axis_index("s") * NC + lax.axis_index("c")
        s0_base = wid * (lt_per_w * LT)
        rbuf = (r0, r1)
        tbuf = (t0, t1)

        stage = []
        for s1 in range(S1):
            stage.append(
                pltpu.async_copy(
                    idx_hbm.at[pl.ds(s1 * S0 + s0_base, lt_per_w * LT)],
                    idx_v.at[pl.ds(s1 * (lt_per_w * LT), lt_per_w * LT)],
                    gsem,
                )
            )
        for cp in stage:
            cp.wait()

        def gather_unit(u, buf):
            return pltpu.async_copy(table_hbm.at[idx_v.at[pl.ds(u * LT, LT)]], buf, gsem)

        def write_unit(u, buf):
            s1 = u >> 2
            cl = u & 3
            return pltpu.async_copy(
                buf.at[:, :, pl.ds(0, LT)],
                out_hbm.at[s1, :, wid * lt_per_w + cl],
                wsem,
            )

        def drain_write():
            pltpu.make_async_copy(
                t0.at[:, :, pl.ds(0, LT)], out_hbm.at[0, :, 0], wsem
            ).wait()

        iota = lax.iota(jnp.int32, VL)
        rs_idx = [((iota + h * VL) >> 3, (iota + h * VL) & 7) for h in range(D // VL)]

        def transpose_unit(rb, tb):
            for h in range(D // VL):
                ri, si = rs_idx[h]

                @plsc.parallel_loop(0, LT, 1, unroll=8)
                def _(t):
                    v = rb[t, pl.ds(h * VL, VL)]
                    col = jnp.full((VL,), 0, jnp.int32) + t
                    plsc.store_scatter(tb, [ri, si, col], v)

        gps = [gather_unit(0, rbuf[0]), gather_unit(1, rbuf[1])]

        def pair_body(p, _):
            for b in range(2):
                u = 2 * p + b
                gps[b].wait()

                @pl.when(u >= 2)
                def _():
                    drain_write()

                transpose_unit(rbuf[b], tbuf[b])

                @pl.when(u + 2 < n_units)
                def _():
                    gather_unit(u + 2, rbuf[b])

                write_unit(u, tbuf[b])
            return 0

        lax.fori_loop(0, n_units // 2, pair_body, 0, unroll=False)
        drain_write()
        drain_write()

    return lookup


@functools.lru_cache(maxsize=None)
def _make_restage(V, D):
    info = plsc.get_sparse_core_info()
    NC, NS, VL = info.num_cores, info.num_subcores, info.num_lanes
    NW = NC * NS
    DR = D // 8
    LT = 128
    NT = V // LT
    TW = V - NT * LT
    n_main = (NT // NW) * NW
    per_w = n_main // NW
    n_tail = NT - n_main
    assert per_w % 2 == 0 and TW % VL == 0 and D == 2 * VL

    mesh = plsc.VectorSubcoreMesh(core_axis_name="c", subcore_axis_name="s")

    @functools.partial(
        pl.kernel,
        mesh=mesh,
        compiler_params=pltpu.CompilerParams(
            use_tc_tiling_on_sc=True, needs_layout_passes=False
        ),
        out_type=jax.ShapeDtypeStruct((V * D,), jnp.float32),
        scratch_types=[
            pltpu.VMEM((DR, 8, LT + 1), jnp.float32),
            pltpu.VMEM((DR, 8, LT + 1), jnp.float32),
            pltpu.VMEM((LT * D,), jnp.float32),
            pltpu.VMEM((LT * D,), jnp.float32),
            pltpu.VMEM((TW * D,), jnp.float32),
            pltpu.SemaphoreType.DMA,
            pltpu.SemaphoreType.DMA,
        ],
    )
    def restage(w3_hbm, wtail_hbm, out_hbm, b0, b1, r0, r1, tb, isem, osem):
        wid = lax.axis_index("s") * NC + lax.axis_index("c")
        bb = (b0, b1)
        rr = (r0, r1)

        def dma_in(t, buf):
            return pltpu.async_copy(
                w3_hbm.at[:, :, pl.ds(t * LT, LT)],
                buf.at[:, :, pl.ds(0, LT)],
                isem,
            )

        def dma_out(t, rbuf):
            return pltpu.async_copy(
                rbuf, out_hbm.at[pl.ds(t * LT * D, LT * D)], osem
            )

        def drain_out():
            pltpu.make_async_copy(r0, out_hbm.at[pl.ds(0, LT * D)], osem).wait()

        iota = lax.iota(jnp.int32, VL)
        rs_idx = [((iota + h * VL) >> 3, (iota + h * VL) & 7) for h in range(D // VL)]

        def transpose(buf, rbuf):
            for h in range(D // VL):
                ri, si = rs_idx[h]

                @plsc.parallel_loop(0, LT, 1, unroll=8)
                def _(l):
                    col = jnp.full((VL,), 0, jnp.int32) + l
                    v = plsc.load_gather(buf, [ri, si, col])
                    rbuf[pl.ds(l * D + h * VL, VL)] = v

        ips = [dma_in(wid, bb[0]), dma_in(wid + NW, bb[1])]

        def pair_body(p, _):
            for b in range(2):
                u = 2 * p + b
                t = wid + u * NW
                ips[b].wait()

                @pl.when(u >= 2)
                def _():
                    drain_out()

                transpose(bb[b], rr[b])

                @pl.when(u + 2 < per_w)
                def _():
                    dma_in(t + 2 * NW, bb[b])

                dma_out(t, rr[b])
            return 0

        lax.fori_loop(0, per_w // 2, pair_body, 0)
        drain_out()
        drain_out()

        @pl.when(wid < n_tail)
        def _():
            t = n_main + wid
            dma_in(t, b0).wait()
            transpose(b0, r0)
            dma_out(t, r0).wait()

        @pl.when(wid == n_tail)
        def _():
            pltpu.async_copy(wtail_hbm, tb, isem).wait()
            pltpu.async_copy(
                tb,
                out_hbm.at[pl.ds(NT * LT * D, TW * D)],
                osem,
            ).wait()

    return restage


def kernel(token_ids, weight):
    S0, S1 = token_ids.shape
    V, D = weight.shape
    tokens_lin = token_ids.T.reshape(S0 * S1)
    w3 = weight.T.reshape(D // 8, 8, V)
    ntail_rows = V - (V // 128) * 128
    wtail = weight[V - ntail_rows :, :].reshape(ntail_rows * D)
    wlin = _make_restage(V, D)(w3, wtail)
    w2 = wlin.reshape(V, D)
    out5 = _make_lookup(S0, S1, V, D)(tokens_lin, w2)
    return out5.transpose(2, 4, 0, 1, 3).reshape(S0, S1, D)


# speedup vs baseline: 1.3878x; 1.3878x over previous
import functools

import jax
import jax.numpy as jnp
from jax import lax
from jax.experimental import pallas as pl
from jax.experimental.pallas import tpu as pltpu
from jax.experimental.pallas import tpu_sc as plsc

_CHUNK = 8192


@functools.lru_cache(maxsize=None)
def _make_lookup(S0, S1, V, D):
    info = plsc.get_sparse_core_info()
    NC, NS, VL = info.num_cores, info.num_subcores, info.num_lanes
    NW = NC * NS
    LT = 128
    DR = D // 8
    n_lt = S0 // LT
    lt_per_w = n_lt // NW
    n_units = S1 * lt_per_w
    b_per_w = n_units * LT
    assert S0 % (LT * NW) == 0 and D % 8 == 0 and n_units % 2 == 0

    mesh = plsc.VectorSubcoreMesh(core_axis_name="c", subcore_axis_name="s")

    @functools.partial(
        pl.kernel,
        mesh=mesh,
        compiler_params=pltpu.CompilerParams(
            use_tc_tiling_on_sc=False, needs_layout_passes=False
        ),
        out_type=jax.ShapeDtypeStruct((S1, DR, n_lt, 8, LT), jnp.float32),
        scratch_types=[
            pltpu.VMEM((b_per_w,), jnp.int32),
            pltpu.VMEM((LT, D), jnp.float32),
            pltpu.VMEM((LT, D), jnp.float32),
            pltpu.VMEM((DR, 8, LT + 1), jnp.float32),
            pltpu.VMEM((DR, 8, LT + 1), jnp.float32),
            pltpu.SemaphoreType.DMA,
            pltpu.SemaphoreType.DMA,
        ],
    )
    def lookup(idx_hbm, table_hbm, out_hbm, idx_v, r0, r1, t0, t1, gsem, wsem):
        wid = lax.axis_index("s") * NC + lax.axis_index("c")
        s0_base = wid * (lt_per_w * LT)
        rbuf = (r0, r1)
        tbuf = (t0, t1)

        stage = []
        for s1 in range(S1):
            stage.append(
                pltpu.async_copy(
                    idx_hbm.at[pl.ds(s1 * S0 + s0_base, lt_per_w * LT)],
                    idx_v.at[pl.ds(s1 * (lt_per_w * LT), lt_per_w * LT)],
                    gsem,
                )
            )
        for cp in stage:
            cp.wait()

        cm = _CHUNK - 1
        qm = _CHUNK // 4 - 1
        qb = _CHUNK.bit_length() - 3

        @plsc.parallel_loop(0, b_per_w // VL, 1, unroll=8)
        def _(g):
            t = idx_v[pl.ds(g * VL, VL)]
            s = (t & ~cm) | ((t & qm) << 2) | ((t & cm) >> qb)
            idx_v[pl.ds(g * VL, VL)] = s

        def gather_unit(u, buf):
            return pltpu.async_copy(table_hbm.at[idx_v.at[pl.ds(u * LT, LT)]], buf, gsem)

        def write_unit(u, buf):
            s1 = u >> 2
            cl = u & 3
            return pltpu.async_copy(
                buf.at[:, :, pl.ds(0, LT)],
                out_hbm.at[s1, :, wid * lt_per_w + cl],
                wsem,
            )

        def drain_write():
            pltpu.make_async_copy(
                t0.at[:, :, pl.ds(0, LT)], out_hbm.at[0, :, 0], wsem
            ).wait()

        iota = lax.iota(jnp.int32, VL)
        rs_idx = [((iota + h * VL) >> 3, (iota + h * VL) & 7) for h in range(D // VL)]

        def transpose_unit(rb, tb):
            for h in range(D // VL):
                ri, si = rs_idx[h]

                @plsc.parallel_loop(0, LT, 1, unroll=8)
                def _(t):
                    v = rb[t, pl.ds(h * VL, VL)]
                    col = jnp.full((VL,), 0, jnp.int32) + t
                    plsc.store_scatter(tb, [ri, si, col], v)

        gps = [gather_unit(0, rbuf[0]), gather_unit(1, rbuf[1])]

        def pair_body(p, _):
            for b in range(2):
                u = 2 * p + b
                gps[b].wait()

                @pl.when(u >= 2)
                def _():
                    drain_write()

                transpose_unit(rbuf[b], tbuf[b])

                @pl.when(u + 2 < n_units)
                def _():
                    gather_unit(u + 2, rbuf[b])

                write_unit(u, tbuf[b])
            return 0

        lax.fori_loop(0, n_units // 2, pair_body, 0, unroll=False)
        drain_write()
        drain_write()

    return lookup


@functools.lru_cache(maxsize=None)
def _make_restage_tc(V, D):
    C = _CHUNK
    Q = C // 4
    OR = C * D // 128
    grid = (V + C - 1) // C
    Vp = grid * C

    def body(w3_ref, out_ref):
        y = w3_ref[...].reshape(D, C)
        for j in range(4):
            out_ref[:, D * j : D * (j + 1)] = y[:, Q * j : Q * (j + 1)].T

    relayout = pl.pallas_call(
        body,
        grid=(grid,),
        in_specs=[pl.BlockSpec((D // 8, 8, C), lambda i: (0, 0, i))],
        out_specs=pl.BlockSpec((OR, 128), lambda i: (i, 0)),
        out_shape=jax.ShapeDtypeStruct((grid * OR, 128), jnp.float32),
    )
    return relayout, Vp


@functools.lru_cache(maxsize=None)
def _make_restage(V, D):
    info = plsc.get_sparse_core_info()
    NC, NS, VL = info.num_cores, info.num_subcores, info.num_lanes
    NW = NC * NS
    DR = D // 8
    LT = 128
    NT = V // LT
    TW = V - NT * LT
    n_main = (NT // NW) * NW
    per_w = n_main // NW
    n_tail = NT - n_main
    assert per_w % 2 == 0 and TW % VL == 0 and D == 2 * VL

    mesh = plsc.VectorSubcoreMesh(core_axis_name="c", subcore_axis_name="s")

    @functools.partial(
        pl.kernel,
        mesh=mesh,
        compiler_params=pltpu.CompilerParams(
            use_tc_tiling_on_sc=True, needs_layout_passes=False
        ),
        out_type=jax.ShapeDtypeStruct((V * D,), jnp.float32),
        scratch_types=[
            pltpu.VMEM((DR, 8, LT + 1), jnp.float32),
            pltpu.VMEM((DR, 8, LT + 1), jnp.float32),
            pltpu.VMEM((LT * D,), jnp.float32),
            pltpu.VMEM((LT * D,), jnp.float32),
            pltpu.VMEM((TW * D,), jnp.float32),
            pltpu.SemaphoreType.DMA,
            pltpu.SemaphoreType.DMA,
        ],
    )
    def restage(w3_hbm, wtail_hbm, out_hbm, b0, b1, r0, r1, tb, isem, osem):
        wid = lax.axis_index("s") * NC + lax.axis_index("c")
        bb = (b0, b1)
        rr = (r0, r1)

        def dma_in(t, buf):
            return pltpu.async_copy(
                w3_hbm.at[:, :, pl.ds(t * LT, LT)],
                buf.at[:, :, pl.ds(0, LT)],
                isem,
            )

        def dma_out(t, rbuf):
            return pltpu.async_copy(
                rbuf, out_hbm.at[pl.ds(t * LT * D, LT * D)], osem
            )

        def drain_out():
            pltpu.make_async_copy(r0, out_hbm.at[pl.ds(0, LT * D)], osem).wait()

        iota = lax.iota(jnp.int32, VL)
        rs_idx = [((iota + h * VL) >> 3, (iota + h * VL) & 7) for h in range(D // VL)]

        def transpose(buf, rbuf):
            for h in range(D // VL):
                ri, si = rs_idx[h]

                @plsc.parallel_loop(0, LT, 1, unroll=8)
                def _(l):
                    col = jnp.full((VL,), 0, jnp.int32) + l
                    v = plsc.load_gather(buf, [ri, si, col])
                    rbuf[pl.ds(l * D + h * VL, VL)] = v

        ips = [dma_in(wid, bb[0]), dma_in(wid + NW, bb[1])]

        def pair_body(p, _):
            for b in range(2):
                u = 2 * p + b
                t = wid + u * NW
                ips[b].wait()

                @pl.when(u >= 2)
                def _():
                    drain_out()

                transpose(bb[b], rr[b])

                @pl.when(u + 2 < per_w)
                def _():
                    dma_in(t + 2 * NW, bb[b])

                dma_out(t, rr[b])
            return 0

        lax.fori_loop(0, per_w // 2, pair_body, 0)
        drain_out()
        drain_out()

        @pl.when(wid < n_tail)
        def _():
            t = n_main + wid
            dma_in(t, b0).wait()
            transpose(b0, r0)
            dma_out(t, r0).wait()

        @pl.when(wid == n_tail)
        def _():
            pltpu.async_copy(wtail_hbm, tb, isem).wait()
            pltpu.async_copy(
                tb,
                out_hbm.at[pl.ds(NT * LT * D, TW * D)],
                osem,
            ).wait()

    return restage


def kernel(token_ids, weight):
    S0, S1 = token_ids.shape
    V, D = weight.shape
    tokens_lin = token_ids.T.reshape(S0 * S1)
    w3 = weight.T.reshape(D // 8, 8, V)
    relayout, Vp = _make_restage_tc(V, D)
    w2 = relayout(w3).reshape(Vp, D)
    out5 = _make_lookup(S0, S1, Vp, D)(tokens_lin, w2)
    return out5.transpose(2, 4, 0, 1, 3).reshape(S0, S1, D)
